# baseline (speedup 1.0000x reference)
import jax
import jax.numpy as jnp
from jax.experimental import pallas as pl
from jax.experimental.pallas import tpu as pltpu

_BM = 80


def _gcn_block(a_ref, e_ref, o_ref):
    o_ref[...] = jnp.dot(a_ref[...], e_ref[...],
                         preferred_element_type=jnp.float32)


def kernel(A_hat, E):
    n, k = A_hat.shape
    d = E.shape[1]
    return pl.pallas_call(
        _gcn_block,
        grid=(n // _BM,),
        in_specs=[
            pl.BlockSpec((_BM, k), lambda i: (i, 0)),
            pl.BlockSpec((k, d), lambda i: (0, 0)),
        ],
        out_specs=pl.BlockSpec((_BM, d), lambda i: (i, 0)),
        out_shape=jax.ShapeDtypeStruct((n, d), jnp.float32),
        compiler_params=pltpu.CompilerParams(
            dimension_semantics=("arbitrary",),
        ),
    )(A_hat, E)


# speedup vs baseline: 1.2976x; 1.2976x over previous
import jax
import jax.numpy as jnp
from jax.experimental import pallas as pl
from jax.experimental.pallas import tpu as pltpu

_BS = 80
_BP = 160


def _gcn_body(a_hbm, e_ref, o_ref, a_buf, sems):
    def pcopy(poff, row):
        return pltpu.make_async_copy(
            a_hbm.at[pl.ds(row, _BP), :],
            a_buf.at[pl.ds(poff, _BP), :],
            sems.at[poff // _BP],
        )

    def lcopy(row):
        return pltpu.make_async_copy(
            a_hbm.at[pl.ds(row, _BS), :],
            a_buf.at[pl.ds(2 * _BP, _BS), :],
            sems.at[2],
        )

    def dot(boff, row):
        o_ref[pl.ds(row, _BS), :] = jnp.dot(
            a_buf[pl.ds(boff, _BS), :], e_ref[...],
            preferred_element_type=jnp.float32)

    pcopy(0, 0).start()
    pcopy(_BP, _BP).start()

    def rotation(r, carry):
        row = 2 * _BP * r
        pcopy(0, row).wait()
        dot(0, row)
        dot(_BS, row + _BS)
        pcopy(0, row + 2 * _BP).start()
        pcopy(_BP, row + _BP).wait()
        dot(_BP, row + _BP)
        dot(_BP + _BS, row + _BP + _BS)
        pcopy(_BP, row + 3 * _BP).start()
        return carry

    n = a_hbm.shape[0]
    nrot = ((n - _BS) // _BP) // 2 - 1
    jax.lax.fori_loop(0, nrot, rotation, 0)
    row = 2 * _BP * nrot
    lcopy(n - _BS).start()
    pcopy(0, row).wait()
    dot(0, row)
    dot(_BS, row + _BS)
    pcopy(_BP, row + _BP).wait()
    dot(_BP, row + _BP)
    dot(_BP + _BS, row + _BP + _BS)
    lcopy(n - _BS).wait()
    dot(2 * _BP, n - _BS)


def kernel(A_hat, E):
    n, k = A_hat.shape
    d = E.shape[1]
    return pl.pallas_call(
        _gcn_body,
        in_specs=[
            pl.BlockSpec(memory_space=pltpu.MemorySpace.HBM),
            pl.BlockSpec(memory_space=pltpu.MemorySpace.VMEM),
        ],
        out_specs=pl.BlockSpec(memory_space=pltpu.MemorySpace.VMEM),
        out_shape=jax.ShapeDtypeStruct((n, d), jnp.float32),
        scratch_shapes=[
            pltpu.MemorySpace.VMEM((2 * _BP + _BS, k), jnp.float32),
            pltpu.SemaphoreType.DMA((3,)),
        ],
    )(A_hat, E)
